# baseline (speedup 1.0000x reference)
import functools

import jax
import jax.numpy as jnp
from jax import lax
from jax.experimental import pallas as pl
from jax.experimental.pallas import tpu as pltpu
from jax.experimental.pallas import tpu_sc as plsc

TOTAL = 32768
D = 2048
B = 16
NC = 2
NS = 16
NW = NC * NS
SC_ROWS = 9216
CHUNK = SC_ROWS // NW
R = 16
NBLK = CHUNK // R
LANES = 16
RB_TC = 512


def _seg_of(cu_scalars, g):
    s = jnp.int32(0)
    for c in cu_scalars:
        s = s + (c <= g).astype(jnp.int32)
    return s


def _stage1_body(flat_hbm, cu_hbm, out_hbm, cu_v, acc, buf, sem):
    wid = lax.axis_index("s") * NC + lax.axis_index("c")
    base_row = wid * CHUNK

    pltpu.sync_copy(cu_hbm.at[pl.ds(0, B)], cu_v)
    cu_vec = cu_v[...]
    cu_scalars = [cu_vec[i] for i in range(1, B)] + [jnp.int32(TOTAL)]

    def zero_body(k, _):
        for jj in range(16):
            acc[pl.ds(k * 256 + jj * LANES, LANES)] = jnp.zeros(
                (LANES,), jnp.float32)
        return 0
    lax.fori_loop(0, (B * D) // 256, zero_body, 0)

    pltpu.async_copy(
        flat_hbm.at[pl.ds(base_row * D, R * D)],
        buf.at[pl.ds(0, R * D)], sem)

    def blk_body(b, _):
        @pl.when(b + 1 < NBLK)
        def _():
            off = (base_row + (b + 1) * R) * D
            slot = ((b + 1) % 2) * R * D
            pltpu.async_copy(flat_hbm.at[pl.ds(off, R * D)],
                             buf.at[pl.ds(slot, R * D)], sem)

        pltpu.make_async_copy(flat_hbm.at[pl.ds(0, R * D)],
                              buf.at[pl.ds(0, R * D)], sem).wait()

        pbase = (b % 2) * R * D
        g0 = base_row + b * R
        segf = _seg_of(cu_scalars, g0)
        segl = _seg_of(cu_scalars, g0 + (R - 1))

        @pl.when(segf == segl)
        def _():
            abase = (segf // 8) * 16384 + (segf % 8) * 128

            def j_body(j, _):
                cbase = abase + j * 1024
                bbase = pbase + j * 1024
                for v in range(8):
                    co = cbase + v * LANES
                    bo = bbase + v * LANES
                    cur = acc[pl.ds(co, LANES)]
                    bs = [buf[pl.ds(bo + t * 16384 + r8 * 128, LANES)]
                          for t in range(2) for r8 in range(8)]
                    while len(bs) > 1:
                        nxt = [bs[i] + bs[i + 1] for i in range(0, len(bs) - 1, 2)]
                        if len(bs) % 2:
                            nxt.append(bs[-1])
                        bs = nxt
                    acc[pl.ds(co, LANES)] = cur + bs[0]
                return 0
            lax.fori_loop(0, D // 128, j_body, 0)

        @pl.when(segf != segl)
        def _():
            for rr in range(R):
                t, r8 = rr // 8, rr % 8
                sr = _seg_of(cu_scalars, g0 + rr)
                ab0 = (sr // 8) * 16384 + (sr % 8) * 128

                def j_body2(j, _, ab0=ab0, t=t, r8=r8):
                    cb = ab0 + j * 1024
                    bb = pbase + j * 1024 + t * 16384 + r8 * 128
                    for v in range(8):
                        acc[pl.ds(cb + v * LANES, LANES)] = (
                            acc[pl.ds(cb + v * LANES, LANES)]
                            + buf[pl.ds(bb + v * LANES, LANES)])
                    return 0
                lax.fori_loop(0, D // 128, j_body2, 0)

        return 0

    lax.fori_loop(0, NBLK, blk_body, 0)

    pltpu.sync_copy(acc, out_hbm.at[pl.ds(wid * B * D, B * D)])


_stage1 = functools.partial(
    pl.kernel,
    out_type=jax.ShapeDtypeStruct((NW * B * D,), jnp.float32),
    mesh=plsc.VectorSubcoreMesh(core_axis_name="c", subcore_axis_name="s",
                                num_cores=NC, num_subcores=NS),
    scratch_types=[
        pltpu.VMEM((B,), jnp.int32),
        pltpu.VMEM((B * D,), jnp.float32),
        pltpu.VMEM((2 * R * D,), jnp.float32),
        pltpu.SemaphoreType.DMA,
    ],
)(_stage1_body)


def _tc_rows_body(cu_ref, x_ref, o_ref):
    i = pl.program_id(0)

    @pl.when(i == 0)
    def _():
        o_ref[...] = jnp.zeros_like(o_ref)

    base = SC_ROWS + i * RB_TC
    segf = jnp.int32(0)
    segl = jnp.int32(0)
    for k in range(1, B):
        segf = segf + (cu_ref[k] <= base).astype(jnp.int32)
        segl = segl + (cu_ref[k] <= base + RB_TC - 1).astype(jnp.int32)

    @pl.when(segf == segl)
    def _():
        ssum = jnp.sum(x_ref[...], axis=0, keepdims=True)
        o_ref[pl.ds(segf, 1), :] = o_ref[pl.ds(segf, 1), :] + ssum

    @pl.when(segf != segl)
    def _():
        sub = lax.broadcasted_iota(jnp.int32, (B, 1), 0)
        lo = jnp.zeros((B, 1), jnp.int32)
        hi = jnp.zeros((B, 1), jnp.int32)
        for k in range(B):
            lo = jnp.where(sub == k, cu_ref[k], lo)
            hi = jnp.where(sub == k, cu_ref[k + 1], hi)
        g = lax.broadcasted_iota(jnp.int32, (B, RB_TC), 1) + base
        oh = ((lo <= g) & (g < hi)).astype(jnp.float32)
        o_ref[...] += lax.dot_general(
            oh, x_ref[...], (((1,), (0,)), ((), ())),
            precision=lax.Precision.HIGHEST,
            preferred_element_type=jnp.float32)


def _combine_body(cu_ref, p_ref, t_ref, o_ref):
    sub = lax.broadcasted_iota(jnp.int32, (B, 1), 0)
    cnt = jnp.zeros((B, 1), jnp.int32)
    for k in range(B):
        cnt = jnp.where(sub == k, cu_ref[k + 1] - cu_ref[k], cnt)
    inv = 1.0 / jnp.maximum(cnt.astype(jnp.float32), 1.0)
    o_ref[...] = (jnp.sum(p_ref[...], axis=0) + t_ref[...]) * inv


def kernel(flat, cu_seqlens):
    flat1d = flat.reshape(TOTAL // 8, 8, D // 128, 128).transpose(
        0, 2, 1, 3).reshape(-1)
    cu = cu_seqlens.astype(jnp.int32)

    partials = _stage1(flat1d, cu)

    tc_part = pl.pallas_call(
        _tc_rows_body,
        grid=((TOTAL - SC_ROWS) // RB_TC,),
        in_specs=[
            pl.BlockSpec(memory_space=pltpu.SMEM),
            pl.BlockSpec((RB_TC, D), lambda i: (SC_ROWS // RB_TC + i, 0)),
        ],
        out_specs=pl.BlockSpec((B, D), lambda i: (0, 0)),
        out_shape=jax.ShapeDtypeStruct((B, D), jnp.float32),
    )(cu, flat)

    p_view = partials.reshape(NW, B // 8, D // 128, 8, 128).transpose(
        0, 1, 3, 2, 4).reshape(NW, B, D)
    out = pl.pallas_call(
        _combine_body,
        in_specs=[
            pl.BlockSpec(memory_space=pltpu.SMEM),
            pl.BlockSpec(memory_space=pltpu.VMEM),
            pl.BlockSpec(memory_space=pltpu.VMEM),
        ],
        out_shape=jax.ShapeDtypeStruct((B, D), jnp.float32),
    )(cu, p_view, tc_part)
    return out


# speedup vs baseline: 1.0347x; 1.0347x over previous
import functools

import jax
import jax.numpy as jnp
from jax import lax
from jax.experimental import pallas as pl
from jax.experimental.pallas import tpu as pltpu
from jax.experimental.pallas import tpu_sc as plsc

TOTAL = 32768
D = 2048
B = 16
NC = 2
NS = 16
NW = NC * NS
SC_ROWS = 9216
CHUNK = SC_ROWS // NW
R = 16
NBLK = CHUNK // R
LANES = 16
RB_TC = 1024


def _seg_of(cu_scalars, g):
    s = jnp.int32(0)
    for c in cu_scalars:
        s = s + (c <= g).astype(jnp.int32)
    return s


def _stage1_body(flat_hbm, cu_hbm, out_hbm, cu_v, acc, buf, sem):
    wid = lax.axis_index("s") * NC + lax.axis_index("c")
    base_row = wid * CHUNK

    pltpu.sync_copy(cu_hbm.at[pl.ds(0, B)], cu_v)
    cu_vec = cu_v[...]
    cu_scalars = [cu_vec[i] for i in range(1, B)] + [jnp.int32(TOTAL)]

    def zero_body(k, _):
        for jj in range(16):
            acc[pl.ds(k * 256 + jj * LANES, LANES)] = jnp.zeros(
                (LANES,), jnp.float32)
        return 0
    lax.fori_loop(0, (B * D) // 256, zero_body, 0)

    pltpu.async_copy(
        flat_hbm.at[pl.ds(base_row * D, R * D)],
        buf.at[pl.ds(0, R * D)], sem)

    def blk_body(b, _):
        @pl.when(b + 1 < NBLK)
        def _():
            off = (base_row + (b + 1) * R) * D
            slot = ((b + 1) % 2) * R * D
            pltpu.async_copy(flat_hbm.at[pl.ds(off, R * D)],
                             buf.at[pl.ds(slot, R * D)], sem)

        pltpu.make_async_copy(flat_hbm.at[pl.ds(0, R * D)],
                              buf.at[pl.ds(0, R * D)], sem).wait()

        pbase = (b % 2) * R * D
        g0 = base_row + b * R
        segf = _seg_of(cu_scalars, g0)
        segl = _seg_of(cu_scalars, g0 + (R - 1))

        @pl.when(segf == segl)
        def _():
            abase = (segf // 8) * 16384 + (segf % 8) * 128

            def j_body(j, _):
                cbase = abase + j * 1024
                bbase = pbase + j * 1024
                for v in range(8):
                    co = cbase + v * LANES
                    bo = bbase + v * LANES
                    cur = acc[pl.ds(co, LANES)]
                    bs = [buf[pl.ds(bo + t * 16384 + r8 * 128, LANES)]
                          for t in range(2) for r8 in range(8)]
                    while len(bs) > 1:
                        nxt = [bs[i] + bs[i + 1] for i in range(0, len(bs) - 1, 2)]
                        if len(bs) % 2:
                            nxt.append(bs[-1])
                        bs = nxt
                    acc[pl.ds(co, LANES)] = cur + bs[0]
                return 0
            lax.fori_loop(0, D // 128, j_body, 0)

        @pl.when(segf != segl)
        def _():
            for rr in range(R):
                t, r8 = rr // 8, rr % 8
                sr = _seg_of(cu_scalars, g0 + rr)
                ab0 = (sr // 8) * 16384 + (sr % 8) * 128

                def j_body2(j, _, ab0=ab0, t=t, r8=r8):
                    cb = ab0 + j * 1024
                    bb = pbase + j * 1024 + t * 16384 + r8 * 128
                    for v in range(8):
                        acc[pl.ds(cb + v * LANES, LANES)] = (
                            acc[pl.ds(cb + v * LANES, LANES)]
                            + buf[pl.ds(bb + v * LANES, LANES)])
                    return 0
                lax.fori_loop(0, D // 128, j_body2, 0)

        return 0

    lax.fori_loop(0, NBLK, blk_body, 0)

    pltpu.sync_copy(acc, out_hbm.at[pl.ds(wid * B * D, B * D)])


_stage1 = functools.partial(
    pl.kernel,
    out_type=jax.ShapeDtypeStruct((NW * B * D,), jnp.float32),
    mesh=plsc.VectorSubcoreMesh(core_axis_name="c", subcore_axis_name="s",
                                num_cores=NC, num_subcores=NS),
    scratch_types=[
        pltpu.VMEM((B,), jnp.int32),
        pltpu.VMEM((B * D,), jnp.float32),
        pltpu.VMEM((2 * R * D,), jnp.float32),
        pltpu.SemaphoreType.DMA,
    ],
)(_stage1_body)


def _tc_rows_body(cu_ref, x_ref, o_ref):
    i = pl.program_id(0)

    @pl.when(i == 0)
    def _():
        o_ref[...] = jnp.zeros_like(o_ref)

    base = SC_ROWS + i * RB_TC
    segf = jnp.int32(0)
    segl = jnp.int32(0)
    for k in range(1, B):
        segf = segf + (cu_ref[k] <= base).astype(jnp.int32)
        segl = segl + (cu_ref[k] <= base + RB_TC - 1).astype(jnp.int32)

    @pl.when(segf == segl)
    def _():
        ssum = jnp.sum(x_ref[...], axis=0, keepdims=True)
        o_ref[pl.ds(segf, 1), :] = o_ref[pl.ds(segf, 1), :] + ssum

    @pl.when(segf != segl)
    def _():
        rows = lax.broadcasted_iota(jnp.int32, (RB_TC, 1), 0) + base
        x = x_ref[...]

        def seg_body(s, _):
            lo_s = cu_ref[s]
            hi_s = cu_ref[s + 1]
            m = ((rows >= lo_s) & (rows < hi_s)).astype(jnp.float32)
            o_ref[pl.ds(s, 1), :] += jnp.sum(x * m, axis=0, keepdims=True)
            return 0
        lax.fori_loop(segf, segl + 1, seg_body, 0)


def _combine_body(cu_ref, p_ref, t_ref, o_ref):
    sub = lax.broadcasted_iota(jnp.int32, (B, 1), 0)
    cnt = jnp.zeros((B, 1), jnp.int32)
    for k in range(B):
        cnt = jnp.where(sub == k, cu_ref[k + 1] - cu_ref[k], cnt)
    inv = 1.0 / jnp.maximum(cnt.astype(jnp.float32), 1.0)
    o_ref[...] = (jnp.sum(p_ref[...], axis=0) + t_ref[...]) * inv


def kernel(flat, cu_seqlens):
    flat1d = flat.reshape(TOTAL // 8, 8, D // 128, 128).transpose(
        0, 2, 1, 3).reshape(-1)
    cu = cu_seqlens.astype(jnp.int32)

    partials = _stage1(flat1d, cu)

    tc_part = pl.pallas_call(
        _tc_rows_body,
        grid=((TOTAL - SC_ROWS) // RB_TC,),
        in_specs=[
            pl.BlockSpec(memory_space=pltpu.SMEM),
            pl.BlockSpec((RB_TC, D), lambda i: (SC_ROWS // RB_TC + i, 0)),
        ],
        out_specs=pl.BlockSpec((B, D), lambda i: (0, 0)),
        out_shape=jax.ShapeDtypeStruct((B, D), jnp.float32),
    )(cu, flat)

    p_view = partials.reshape(NW, B // 8, D // 128, 8, 128).transpose(
        0, 1, 3, 2, 4).reshape(NW, B, D)
    out = pl.pallas_call(
        _combine_body,
        in_specs=[
            pl.BlockSpec(memory_space=pltpu.SMEM),
            pl.BlockSpec(memory_space=pltpu.VMEM),
            pl.BlockSpec(memory_space=pltpu.VMEM),
        ],
        out_shape=jax.ShapeDtypeStruct((B, D), jnp.float32),
    )(cu, p_view, tc_part)
    return out


# speedup vs baseline: 1.0623x; 1.0267x over previous
import functools

import jax
import jax.numpy as jnp
from jax import lax
from jax.experimental import pallas as pl
from jax.experimental.pallas import tpu as pltpu
from jax.experimental.pallas import tpu_sc as plsc

TOTAL = 32768
D = 2048
B = 16
NC = 2
NS = 16
NW = NC * NS
SC_ROWS = 9216
CHUNK = SC_ROWS // NW
R = 16
NBLK = CHUNK // R
LANES = 16
RB_TC = 1024


def _seg_of(cu_scalars, g):
    s = jnp.int32(0)
    for c in cu_scalars:
        s = s + (c <= g).astype(jnp.int32)
    return s


def _stage1_body(flat_hbm, cu_hbm, out_hbm, cu_v, acc, buf, sem):
    wid = lax.axis_index("s") * NC + lax.axis_index("c")
    base_row = wid * CHUNK

    pltpu.sync_copy(cu_hbm.at[pl.ds(0, B)], cu_v)
    cu_vec = cu_v[...]
    cu_scalars = [cu_vec[i] for i in range(1, B)] + [jnp.int32(TOTAL)]

    def zero_body(k, _):
        for jj in range(16):
            acc[pl.ds(k * 256 + jj * LANES, LANES)] = jnp.zeros(
                (LANES,), jnp.float32)
        return 0
    lax.fori_loop(0, (B * D) // 256, zero_body, 0)

    pltpu.async_copy(
        flat_hbm.at[pl.ds(base_row * D, R * D)],
        buf.at[pl.ds(0, R * D)], sem)

    def blk_body(b, _):
        @pl.when(b + 1 < NBLK)
        def _():
            off = (base_row + (b + 1) * R) * D
            slot = ((b + 1) % 2) * R * D
            pltpu.async_copy(flat_hbm.at[pl.ds(off, R * D)],
                             buf.at[pl.ds(slot, R * D)], sem)

        pltpu.make_async_copy(flat_hbm.at[pl.ds(0, R * D)],
                              buf.at[pl.ds(0, R * D)], sem).wait()

        pbase = (b % 2) * R * D
        g0 = base_row + b * R
        segf = _seg_of(cu_scalars, g0)
        segl = _seg_of(cu_scalars, g0 + (R - 1))

        @pl.when(segf == segl)
        def _():
            abase = (segf // 8) * 16384 + (segf % 8) * 128

            def j_body(j, _):
                cbase = abase + j * 1024
                bbase = pbase + j * 1024
                for v in range(8):
                    co = cbase + v * LANES
                    bo = bbase + v * LANES
                    cur = acc[pl.ds(co, LANES)]
                    bs = [buf[pl.ds(bo + t * 16384 + r8 * 128, LANES)]
                          for t in range(2) for r8 in range(8)]
                    while len(bs) > 1:
                        nxt = [bs[i] + bs[i + 1] for i in range(0, len(bs) - 1, 2)]
                        if len(bs) % 2:
                            nxt.append(bs[-1])
                        bs = nxt
                    acc[pl.ds(co, LANES)] = cur + bs[0]
                return 0
            lax.fori_loop(0, D // 128, j_body, 0)

        @pl.when(segf != segl)
        def _():
            for rr in range(R):
                t, r8 = rr // 8, rr % 8
                sr = _seg_of(cu_scalars, g0 + rr)
                ab0 = (sr // 8) * 16384 + (sr % 8) * 128

                def j_body2(j, _, ab0=ab0, t=t, r8=r8):
                    cb = ab0 + j * 1024
                    bb = pbase + j * 1024 + t * 16384 + r8 * 128
                    for v in range(8):
                        acc[pl.ds(cb + v * LANES, LANES)] = (
                            acc[pl.ds(cb + v * LANES, LANES)]
                            + buf[pl.ds(bb + v * LANES, LANES)])
                    return 0
                lax.fori_loop(0, D // 128, j_body2, 0)

        return 0

    lax.fori_loop(0, NBLK, blk_body, 0)

    pltpu.sync_copy(acc, out_hbm.at[pl.ds(wid * B * D, B * D)])


_stage1 = functools.partial(
    pl.kernel,
    out_type=jax.ShapeDtypeStruct((NW * B * D,), jnp.float32),
    mesh=plsc.VectorSubcoreMesh(core_axis_name="c", subcore_axis_name="s",
                                num_cores=NC, num_subcores=NS),
    scratch_types=[
        pltpu.VMEM((B,), jnp.int32),
        pltpu.VMEM((B * D,), jnp.float32),
        pltpu.VMEM((2 * R * D,), jnp.float32),
        pltpu.SemaphoreType.DMA,
    ],
)(_stage1_body)


def _tc_rows_body(cu_ref, x_ref, o_ref):
    i = pl.program_id(0)

    @pl.when(i == 0)
    def _():
        o_ref[...] = jnp.zeros_like(o_ref)

    base = SC_ROWS + i * RB_TC
    segf = jnp.int32(0)
    segl = jnp.int32(0)
    for k in range(1, B):
        segf = segf + (cu_ref[k] <= base).astype(jnp.int32)
        segl = segl + (cu_ref[k] <= base + RB_TC - 1).astype(jnp.int32)

    @pl.when(segf == segl)
    def _():
        ssum = jnp.sum(x_ref[...], axis=0, keepdims=True)
        o_ref[pl.ds(segf, 1), :] = o_ref[pl.ds(segf, 1), :] + ssum

    @pl.when(segf != segl)
    def _():
        RSB = 128
        for sb in range(RB_TC // RSB):
            b2 = base + sb * RSB
            sf = jnp.int32(0)
            sl = jnp.int32(0)
            for k in range(1, B):
                sf = sf + (cu_ref[k] <= b2).astype(jnp.int32)
                sl = sl + (cu_ref[k] <= b2 + RSB - 1).astype(jnp.int32)

            @pl.when(sf == sl)
            def _(sb=sb, sf=sf):
                ss = jnp.sum(x_ref[pl.ds(sb * RSB, RSB), :], axis=0,
                             keepdims=True)
                o_ref[pl.ds(sf, 1), :] += ss

            @pl.when(sf != sl)
            def _(sb=sb, b2=b2, sf=sf, sl=sl):
                rows = lax.broadcasted_iota(jnp.int32, (RSB, 1), 0) + b2
                xs = x_ref[pl.ds(sb * RSB, RSB), :]

                def seg_body(s, _):
                    m = ((rows >= cu_ref[s]) & (rows < cu_ref[s + 1])
                         ).astype(jnp.float32)
                    o_ref[pl.ds(s, 1), :] += jnp.sum(
                        xs * m, axis=0, keepdims=True)
                    return 0
                lax.fori_loop(sf, sl + 1, seg_body, 0)


def _combine_body(cu_ref, p_ref, t_ref, o_ref):
    sub = lax.broadcasted_iota(jnp.int32, (B, 1), 0)
    cnt = jnp.zeros((B, 1), jnp.int32)
    for k in range(B):
        cnt = jnp.where(sub == k, cu_ref[k + 1] - cu_ref[k], cnt)
    inv = 1.0 / jnp.maximum(cnt.astype(jnp.float32), 1.0)
    o_ref[...] = (jnp.sum(p_ref[...], axis=0) + t_ref[...]) * inv


def kernel(flat, cu_seqlens):
    flat1d = flat.reshape(TOTAL // 8, 8, D // 128, 128).transpose(
        0, 2, 1, 3).reshape(-1)
    cu = cu_seqlens.astype(jnp.int32)

    partials = _stage1(flat1d, cu)

    tc_part = pl.pallas_call(
        _tc_rows_body,
        grid=((TOTAL - SC_ROWS) // RB_TC,),
        in_specs=[
            pl.BlockSpec(memory_space=pltpu.SMEM),
            pl.BlockSpec((RB_TC, D), lambda i: (SC_ROWS // RB_TC + i, 0)),
        ],
        out_specs=pl.BlockSpec((B, D), lambda i: (0, 0)),
        out_shape=jax.ShapeDtypeStruct((B, D), jnp.float32),
    )(cu, flat)

    p_view = partials.reshape(NW, B // 8, D // 128, 8, 128).transpose(
        0, 1, 3, 2, 4).reshape(NW, B, D)
    out = pl.pallas_call(
        _combine_body,
        in_specs=[
            pl.BlockSpec(memory_space=pltpu.SMEM),
            pl.BlockSpec(memory_space=pltpu.VMEM),
            pl.BlockSpec(memory_space=pltpu.VMEM),
        ],
        out_shape=jax.ShapeDtypeStruct((B, D), jnp.float32),
    )(cu, p_view, tc_part)
    return out


# speedup vs baseline: 1.0649x; 1.0024x over previous
import functools

import jax
import jax.numpy as jnp
from jax import lax
from jax.experimental import pallas as pl
from jax.experimental.pallas import tpu as pltpu
from jax.experimental.pallas import tpu_sc as plsc

TOTAL = 32768
D = 2048
B = 16
NC = 2
NS = 16
NW = NC * NS
SC_ROWS = 9216
CHUNK = SC_ROWS // NW
R = 16
NBLK = CHUNK // R
LANES = 16
RB_TC = 1024


def _seg_of(cu_scalars, g):
    s = jnp.int32(0)
    for c in cu_scalars:
        s = s + (c <= g).astype(jnp.int32)
    return s


def _stage1_body(flat_hbm, cu_hbm, out_hbm, cu_v, acc, buf, sem):
    wid = lax.axis_index("s") * NC + lax.axis_index("c")
    base_row = wid * CHUNK

    pltpu.sync_copy(cu_hbm.at[pl.ds(0, B)], cu_v)
    cu_vec = cu_v[...]
    cu_scalars = [cu_vec[i] for i in range(1, B)] + [jnp.int32(TOTAL)]

    def zero_body(k, _):
        for jj in range(16):
            acc[pl.ds(k * 256 + jj * LANES, LANES)] = jnp.zeros(
                (LANES,), jnp.float32)
        return 0
    lax.fori_loop(0, (B * D) // 256, zero_body, 0)

    pltpu.async_copy(
        flat_hbm.at[pl.ds(base_row * D, R * D)],
        buf.at[pl.ds(0, R * D)], sem)

    def blk_body(b, _):
        @pl.when(b + 1 < NBLK)
        def _():
            off = (base_row + (b + 1) * R) * D
            slot = ((b + 1) % 2) * R * D
            pltpu.async_copy(flat_hbm.at[pl.ds(off, R * D)],
                             buf.at[pl.ds(slot, R * D)], sem)

        pltpu.make_async_copy(flat_hbm.at[pl.ds(0, R * D)],
                              buf.at[pl.ds(0, R * D)], sem).wait()

        pbase = (b % 2) * R * D
        g0 = base_row + b * R
        segf = _seg_of(cu_scalars, g0)
        segl = _seg_of(cu_scalars, g0 + (R - 1))

        @pl.when(segf == segl)
        def _():
            abase = (segf // 8) * 16384 + (segf % 8) * 128

            def j_body(j, _):
                cbase = abase + j * 1024
                bbase = pbase + j * 1024
                for v in range(8):
                    co = cbase + v * LANES
                    bo = bbase + v * LANES
                    cur = acc[pl.ds(co, LANES)]
                    bs = [buf[pl.ds(bo + t * 16384 + r8 * 128, LANES)]
                          for t in range(2) for r8 in range(8)]
                    while len(bs) > 1:
                        nxt = [bs[i] + bs[i + 1] for i in range(0, len(bs) - 1, 2)]
                        if len(bs) % 2:
                            nxt.append(bs[-1])
                        bs = nxt
                    acc[pl.ds(co, LANES)] = cur + bs[0]
                return 0
            lax.fori_loop(0, D // 128, j_body, 0)

        @pl.when(segf != segl)
        def _():
            def row_body(rr, _):
                t = rr // 8
                r8 = rr % 8
                sr = _seg_of(cu_scalars, g0 + rr)
                ab0 = (sr // 8) * 16384 + (sr % 8) * 128
                rbase = pbase + t * 16384 + r8 * 128

                def j_body2(j, _):
                    cb = ab0 + j * 1024
                    bb = rbase + j * 1024
                    for v in range(8):
                        acc[pl.ds(cb + v * LANES, LANES)] = (
                            acc[pl.ds(cb + v * LANES, LANES)]
                            + buf[pl.ds(bb + v * LANES, LANES)])
                    return 0
                lax.fori_loop(0, D // 128, j_body2, 0)
                return 0
            lax.fori_loop(0, R, row_body, 0)

        return 0

    lax.fori_loop(0, NBLK, blk_body, 0)

    pltpu.sync_copy(acc, out_hbm.at[pl.ds(wid * B * D, B * D)])


_stage1 = functools.partial(
    pl.kernel,
    out_type=jax.ShapeDtypeStruct((NW * B * D,), jnp.float32),
    mesh=plsc.VectorSubcoreMesh(core_axis_name="c", subcore_axis_name="s",
                                num_cores=NC, num_subcores=NS),
    scratch_types=[
        pltpu.VMEM((B,), jnp.int32),
        pltpu.VMEM((B * D,), jnp.float32),
        pltpu.VMEM((2 * R * D,), jnp.float32),
        pltpu.SemaphoreType.DMA,
    ],
)(_stage1_body)


def _tc_rows_body(cu_ref, x_ref, o_ref):
    i = pl.program_id(0)

    @pl.when(i == 0)
    def _():
        o_ref[...] = jnp.zeros_like(o_ref)

    base = SC_ROWS + i * RB_TC
    segf = jnp.int32(0)
    segl = jnp.int32(0)
    for k in range(1, B):
        segf = segf + (cu_ref[k] <= base).astype(jnp.int32)
        segl = segl + (cu_ref[k] <= base + RB_TC - 1).astype(jnp.int32)

    @pl.when(segf == segl)
    def _():
        ssum = jnp.sum(x_ref[...], axis=0, keepdims=True)
        o_ref[pl.ds(segf, 1), :] = o_ref[pl.ds(segf, 1), :] + ssum

    @pl.when(segf != segl)
    def _():
        RSB = 128
        for sb in range(RB_TC // RSB):
            b2 = base + sb * RSB
            sf = jnp.int32(0)
            sl = jnp.int32(0)
            for k in range(1, B):
                sf = sf + (cu_ref[k] <= b2).astype(jnp.int32)
                sl = sl + (cu_ref[k] <= b2 + RSB - 1).astype(jnp.int32)

            @pl.when(sf == sl)
            def _(sb=sb, sf=sf):
                ss = jnp.sum(x_ref[pl.ds(sb * RSB, RSB), :], axis=0,
                             keepdims=True)
                o_ref[pl.ds(sf, 1), :] += ss

            @pl.when(sf != sl)
            def _(sb=sb, b2=b2, sf=sf, sl=sl):
                rows = lax.broadcasted_iota(jnp.int32, (RSB, 1), 0) + b2
                xs = x_ref[pl.ds(sb * RSB, RSB), :]

                def seg_body(s, _):
                    m = ((rows >= cu_ref[s]) & (rows < cu_ref[s + 1])
                         ).astype(jnp.float32)
                    o_ref[pl.ds(s, 1), :] += jnp.sum(
                        xs * m, axis=0, keepdims=True)
                    return 0
                lax.fori_loop(sf, sl + 1, seg_body, 0)


def _combine_body(cu_ref, p_ref, t_ref, o_ref):
    sub = lax.broadcasted_iota(jnp.int32, (B, 1), 0)
    cnt = jnp.zeros((B, 1), jnp.int32)
    for k in range(B):
        cnt = jnp.where(sub == k, cu_ref[k + 1] - cu_ref[k], cnt)
    inv = 1.0 / jnp.maximum(cnt.astype(jnp.float32), 1.0)
    o_ref[...] = (jnp.sum(p_ref[...], axis=0) + t_ref[...]) * inv


def kernel(flat, cu_seqlens):
    flat1d = flat.reshape(TOTAL // 8, 8, D // 128, 128).transpose(
        0, 2, 1, 3).reshape(-1)
    cu = cu_seqlens.astype(jnp.int32)

    partials = _stage1(flat1d, cu)

    tc_part = pl.pallas_call(
        _tc_rows_body,
        grid=((TOTAL - SC_ROWS) // RB_TC,),
        in_specs=[
            pl.BlockSpec(memory_space=pltpu.SMEM),
            pl.BlockSpec((RB_TC, D), lambda i: (SC_ROWS // RB_TC + i, 0)),
        ],
        out_specs=pl.BlockSpec((B, D), lambda i: (0, 0)),
        out_shape=jax.ShapeDtypeStruct((B, D), jnp.float32),
    )(cu, flat)

    p_view = partials.reshape(NW, B // 8, D // 128, 8, 128).transpose(
        0, 1, 3, 2, 4).reshape(NW, B, D)
    out = pl.pallas_call(
        _combine_body,
        in_specs=[
            pl.BlockSpec(memory_space=pltpu.SMEM),
            pl.BlockSpec(memory_space=pltpu.VMEM),
            pl.BlockSpec(memory_space=pltpu.VMEM),
        ],
        out_shape=jax.ShapeDtypeStruct((B, D), jnp.float32),
    )(cu, p_view, tc_part)
    return out
